# baseline (speedup 1.0000x reference)
import functools

import jax
import jax.numpy as jnp
from jax import lax
from jax.experimental import pallas as pl
from jax.experimental.pallas import tpu as pltpu
from jax.experimental.pallas import tpu_sc as plsc

B = 16384
S = 20
DV = 6
DD = 16
SV = 12
SD = 16
NUM = 3
OUT = DD + SD + NUM

NC = 2
NS = 16
NW = NC * NS
BW = B // NW
L = 16
NG = BW // L
PV = SV * SV
PREPW = PV * SD + 1
DREPW = DV * DD + 1


def _splat_i32(v):
    return jnp.full((L,), v, jnp.int32)


def _tree_sum(vals):
    while len(vals) > 1:
        vals = [a + b for a, b in zip(vals[::2], vals[1::2])] + (
            [vals[-1]] if len(vals) % 2 else [])
    return vals[0]


def _body(dmg_hbm, spec_hbm, num_hbm, dtab_hbm, stab_hbm, out_hbm,
          dmg_v, spec_v, num_v, dtab_v, stab_v, out_v, prep_v, drep_v):
    cid = lax.axis_index("c")
    sid = lax.axis_index("s")
    wid = sid * NC + cid
    base = wid * BW

    pltpu.sync_copy(dmg_hbm.at[pl.ds(base, BW)], dmg_v)
    pltpu.sync_copy(spec_hbm.at[:, pl.ds(base, BW)], spec_v)
    pltpu.sync_copy(num_hbm.at[pl.ds(base, BW)], num_v)
    pltpu.sync_copy(dtab_hbm, dtab_v)
    pltpu.sync_copy(stab_hbm, stab_v)

    lane = lax.iota(jnp.int32, L)
    inv_s = jnp.full((L,), 1.0 / S, jnp.float32)

    for a in range(DV):
        row = dtab_v[a, pl.ds(0, DD)]
        for l in range(L):
            drep_v[l, pl.ds(a * DD, DD)] = row
    for a in range(SV):
        ra = stab_v[a, pl.ds(0, SD)]
        for b in range(SV):
            row = ra + stab_v[b, pl.ds(0, SD)]
            off = (a * SV + b) * SD
            for l in range(L):
                prep_v[l, pl.ds(off, SD)] = row

    def group(g, carry):
        bidx = g * L + lane
        dv16 = dmg_v[pl.ds(g * L, L)] * DD
        for d in range(DD):
            vals = plsc.load_gather(drep_v, [lane, dv16 + d])
            plsc.store_scatter(out_v, [bidx, _splat_i32(d)], vals)
        sidx = [spec_v[s, pl.ds(g * L, L)] for s in range(S)]
        pidx16 = [(sidx[2 * t] * SV + sidx[2 * t + 1]) * SD
                  for t in range(S // 2)]
        for d in range(SD):
            acc = _tree_sum([plsc.load_gather(prep_v, [lane, p + d])
                             for p in pidx16])
            plsc.store_scatter(out_v, [bidx, _splat_i32(DD + d)], acc * inv_s)
        for j in range(NUM):
            vals = plsc.load_gather(num_v, [bidx, _splat_i32(j)])
            plsc.store_scatter(out_v, [bidx, _splat_i32(DD + SD + j)], vals)
        return carry

    lax.fori_loop(0, NG, group, 0)

    pltpu.sync_copy(out_v, out_hbm.at[pl.ds(base, BW)])


@jax.jit
def _encode(damage_type_ids, special_indices_t, numerical, damage_table,
            special_table):
    mesh = plsc.VectorSubcoreMesh(core_axis_name="c", subcore_axis_name="s")
    run = functools.partial(
        pl.kernel,
        mesh=mesh,
        out_type=jax.ShapeDtypeStruct((B, OUT), jnp.float32),
        compiler_params=pltpu.CompilerParams(needs_layout_passes=False,
                                             use_tc_tiling_on_sc=False),
        scratch_types=[
            pltpu.VMEM((BW,), jnp.int32),
            pltpu.VMEM((S, BW), jnp.int32),
            pltpu.VMEM((BW, NUM), jnp.float32),
            pltpu.VMEM((DV, DD), jnp.float32),
            pltpu.VMEM((SV, SD), jnp.float32),
            pltpu.VMEM((BW, OUT), jnp.float32),
            pltpu.VMEM((L, PREPW), jnp.float32),
            pltpu.VMEM((L, DREPW), jnp.float32),
        ],
    )(_body)
    return run(damage_type_ids, special_indices_t, numerical, damage_table,
               special_table)


def kernel(damage_type_ids, special_indices, numerical, damage_table,
           special_table):
    return _encode(damage_type_ids.astype(jnp.int32),
                   special_indices.astype(jnp.int32).T,
                   numerical, damage_table, special_table)


# speedup vs baseline: 1.0830x; 1.0830x over previous
import functools

import jax
import jax.numpy as jnp
from jax import lax
from jax.experimental import pallas as pl
from jax.experimental.pallas import tpu as pltpu
from jax.experimental.pallas import tpu_sc as plsc

B = 16384
S = 20
DV = 6
DD = 16
SV = 12
SD = 16
NUM = 3
OUT = DD + SD + NUM

NC = 2
NS = 16
NW = NC * NS
BW = B // NW
L = 16
NG = BW // L
PV = SV * SV
PREPW = PV * SD + 1
DREPW = DV * DD + 1


def _splat_i32(v):
    return jnp.full((L,), v, jnp.int32)


def _tree_sum(vals):
    while len(vals) > 1:
        vals = [a + b for a, b in zip(vals[::2], vals[1::2])] + (
            [vals[-1]] if len(vals) % 2 else [])
    return vals[0]


def _body(dmg_hbm, spec_hbm, num_hbm, dtab_hbm, stab_hbm, out_hbm,
          dmg_v, spec_v, num_v, dtab_v, stab_v, out_v, prep_v, drep_v):
    cid = lax.axis_index("c")
    sid = lax.axis_index("s")
    wid = sid * NC + cid
    base = wid * BW

    pltpu.sync_copy(dmg_hbm.at[pl.ds(base, BW)], dmg_v)
    pltpu.sync_copy(spec_hbm.at[:, pl.ds(base, BW)], spec_v)
    pltpu.sync_copy(num_hbm.at[pl.ds(base, BW)], num_v)
    pltpu.sync_copy(dtab_hbm, dtab_v)
    pltpu.sync_copy(stab_hbm, stab_v)

    lane = lax.iota(jnp.int32, L)
    inv_s = jnp.full((L,), 1.0 / S, jnp.float32)

    for a in range(DV):
        row = dtab_v[a, pl.ds(0, DD)]
        for l in range(L):
            drep_v[l, pl.ds(a * DD, DD)] = row
    for a in range(SV):
        ra = stab_v[a, pl.ds(0, SD)]
        for b in range(SV):
            row = ra + stab_v[b, pl.ds(0, SD)]
            off = (a * SV + b) * SD
            for l in range(L):
                prep_v[l, pl.ds(off, SD)] = row

    @plsc.parallel_loop(0, NG, 1, unroll=1)
    def group(g):
        bidx = g * L + lane
        dv16 = dmg_v[pl.ds(g * L, L)] * DD
        for d in range(DD):
            vals = plsc.load_gather(drep_v, [lane, dv16 + d])
            plsc.store_scatter(out_v, [bidx, _splat_i32(d)], vals)
        sidx = [spec_v[s, pl.ds(g * L, L)] for s in range(S)]
        pidx16 = [(sidx[2 * t] * SV + sidx[2 * t + 1]) * SD
                  for t in range(S // 2)]
        for d in range(SD):
            acc = _tree_sum([plsc.load_gather(prep_v, [lane, p + d])
                             for p in pidx16])
            plsc.store_scatter(out_v, [bidx, _splat_i32(DD + d)], acc * inv_s)
        for j in range(NUM):
            vals = plsc.load_gather(num_v, [bidx, _splat_i32(j)])
            plsc.store_scatter(out_v, [bidx, _splat_i32(DD + SD + j)], vals)

    pltpu.sync_copy(out_v, out_hbm.at[pl.ds(base, BW)])


@jax.jit
def _encode(damage_type_ids, special_indices_t, numerical, damage_table,
            special_table):
    mesh = plsc.VectorSubcoreMesh(core_axis_name="c", subcore_axis_name="s")
    run = functools.partial(
        pl.kernel,
        mesh=mesh,
        out_type=jax.ShapeDtypeStruct((B, OUT), jnp.float32),
        compiler_params=pltpu.CompilerParams(needs_layout_passes=False,
                                             use_tc_tiling_on_sc=False),
        scratch_types=[
            pltpu.VMEM((BW,), jnp.int32),
            pltpu.VMEM((S, BW), jnp.int32),
            pltpu.VMEM((BW, NUM), jnp.float32),
            pltpu.VMEM((DV, DD), jnp.float32),
            pltpu.VMEM((SV, SD), jnp.float32),
            pltpu.VMEM((BW, OUT), jnp.float32),
            pltpu.VMEM((L, PREPW), jnp.float32),
            pltpu.VMEM((L, DREPW), jnp.float32),
        ],
    )(_body)
    return run(damage_type_ids, special_indices_t, numerical, damage_table,
               special_table)


def kernel(damage_type_ids, special_indices, numerical, damage_table,
           special_table):
    return _encode(damage_type_ids.astype(jnp.int32),
                   special_indices.astype(jnp.int32).T,
                   numerical, damage_table, special_table)


# speedup vs baseline: 1.1008x; 1.0164x over previous
import functools

import jax
import jax.numpy as jnp
from jax import lax
from jax.experimental import pallas as pl
from jax.experimental.pallas import tpu as pltpu
from jax.experimental.pallas import tpu_sc as plsc

B = 16384
S = 20
DV = 6
DD = 16
SV = 12
SD = 16
NUM = 3
OUT = DD + SD + NUM

NC = 2
NS = 16
NW = NC * NS
BW = B // NW
L = 16
NG = BW // L
PV = SV * SV
PREPW = PV * SD + 1
DREPW = DV * DD + 1


def _splat_i32(v):
    return jnp.full((L,), v, jnp.int32)


def _tree_sum(vals):
    while len(vals) > 1:
        vals = [a + b for a, b in zip(vals[::2], vals[1::2])] + (
            [vals[-1]] if len(vals) % 2 else [])
    return vals[0]


def _body(dmg_hbm, spec_hbm, num_hbm, dtab_hbm, stab_hbm, out_hbm,
          dmg_v, spec_v, num_v, dtab_v, stab_v, out_v, prep_v, drep_v,
          sem0, sem1, sem2, sem3, sem4):
    cid = lax.axis_index("c")
    sid = lax.axis_index("s")
    wid = sid * NC + cid
    base = wid * BW

    c_dtab = pltpu.async_copy(dtab_hbm, dtab_v, sem3)
    c_stab = pltpu.async_copy(stab_hbm, stab_v, sem4)
    c_dmg = pltpu.async_copy(dmg_hbm.at[pl.ds(base, BW)], dmg_v, sem0)
    c_spec = pltpu.async_copy(spec_hbm.at[:, pl.ds(base, BW)], spec_v, sem1)
    c_num = pltpu.async_copy(num_hbm.at[pl.ds(base, BW)], num_v, sem2)
    c_dtab.wait()
    c_stab.wait()

    lane = lax.iota(jnp.int32, L)
    inv_s = jnp.full((L,), 1.0 / S, jnp.float32)

    for a in range(DV):
        row = dtab_v[a, pl.ds(0, DD)]
        for l in range(L):
            drep_v[l, pl.ds(a * DD, DD)] = row
    for a in range(SV):
        ra = stab_v[a, pl.ds(0, SD)]
        for b in range(SV):
            row = ra + stab_v[b, pl.ds(0, SD)]
            off = (a * SV + b) * SD
            for l in range(L):
                prep_v[l, pl.ds(off, SD)] = row

    c_dmg.wait()
    c_spec.wait()
    c_num.wait()

    @plsc.parallel_loop(0, NG, 1, unroll=2)
    def group(g):
        bidx = g * L + lane
        dv16 = dmg_v[pl.ds(g * L, L)] * DD
        for d in range(DD):
            vals = plsc.load_gather(drep_v, [lane, dv16 + d])
            plsc.store_scatter(out_v, [bidx, _splat_i32(d)], vals)
        sidx = [spec_v[s, pl.ds(g * L, L)] for s in range(S)]
        pidx16 = [(sidx[2 * t] * SV + sidx[2 * t + 1]) * SD
                  for t in range(S // 2)]
        for d in range(SD):
            acc = _tree_sum([plsc.load_gather(prep_v, [lane, p + d])
                             for p in pidx16])
            plsc.store_scatter(out_v, [bidx, _splat_i32(DD + d)], acc * inv_s)
        for j in range(NUM):
            vals = plsc.load_gather(num_v, [bidx, _splat_i32(j)])
            plsc.store_scatter(out_v, [bidx, _splat_i32(DD + SD + j)], vals)

    pltpu.sync_copy(out_v, out_hbm.at[pl.ds(base, BW)])


@jax.jit
def _encode(damage_type_ids, special_indices_t, numerical, damage_table,
            special_table):
    mesh = plsc.VectorSubcoreMesh(core_axis_name="c", subcore_axis_name="s")
    run = functools.partial(
        pl.kernel,
        mesh=mesh,
        out_type=jax.ShapeDtypeStruct((B, OUT), jnp.float32),
        compiler_params=pltpu.CompilerParams(needs_layout_passes=False,
                                             use_tc_tiling_on_sc=False),
        scratch_types=[
            pltpu.VMEM((BW,), jnp.int32),
            pltpu.VMEM((S, BW), jnp.int32),
            pltpu.VMEM((BW, NUM), jnp.float32),
            pltpu.VMEM((DV, DD), jnp.float32),
            pltpu.VMEM((SV, SD), jnp.float32),
            pltpu.VMEM((BW, OUT), jnp.float32),
            pltpu.VMEM((L, PREPW), jnp.float32),
            pltpu.VMEM((L, DREPW), jnp.float32),
            pltpu.SemaphoreType.DMA,
            pltpu.SemaphoreType.DMA,
            pltpu.SemaphoreType.DMA,
            pltpu.SemaphoreType.DMA,
            pltpu.SemaphoreType.DMA,
        ],
    )(_body)
    return run(damage_type_ids, special_indices_t, numerical, damage_table,
               special_table)


def kernel(damage_type_ids, special_indices, numerical, damage_table,
           special_table):
    return _encode(damage_type_ids.astype(jnp.int32),
                   special_indices.astype(jnp.int32).T,
                   numerical, damage_table, special_table)
